# baseline (speedup 1.0000x reference)
import functools

import jax
import jax.numpy as jnp
from jax import lax
from jax.experimental import pallas as pl
from jax.experimental.pallas import tpu as pltpu
from jax.experimental.pallas import tpu_sc as plsc

NC = 2
NS = 16
NW = NC * NS
L = 16

D = 64
N_NEG = 20
C = 64


def _sc_dots(B):
  BPW = B // NW
  G = BPW // C
  NROWS = C * N_NEG
  NIDX_R = NROWS // 128

  mesh = plsc.VectorSubcoreMesh(
      core_axis_name="c", subcore_axis_name="s",
      num_cores=NC, num_subcores=NS)

  @functools.partial(
      pl.kernel,
      out_type=[
          jax.ShapeDtypeStruct((B,), jnp.float32),
          jax.ShapeDtypeStruct((N_NEG * B,), jnp.float32),
      ],
      mesh=mesh,
      compiler_params=pltpu.CompilerParams(needs_layout_passes=False,
                                           use_tc_tiling_on_sc=False),
      scratch_types=[
          pltpu.VMEM((C,), jnp.int32),
          pltpu.VMEM((C,), jnp.int32),
          pltpu.VMEM((NIDX_R, 128), jnp.int32),
          pltpu.VMEM((C, D), jnp.float32),
          pltpu.VMEM((C, D), jnp.float32),
          pltpu.VMEM((NROWS, D), jnp.float32),
          pltpu.VMEM((C,), jnp.float32),
          pltpu.VMEM((N_NEG * C,), jnp.float32),
          pltpu.SemaphoreType.DMA,
      ],
  )
  def sc_kernel(center_hbm, context_hbm, negidx_hbm, ctab_hbm, xtab_hbm,
                out1_hbm, out2_hbm,
                cidx_v, xidx_v, nidx_v, crow_v, xrow_v, nrow_v,
                o1_v, o2_v, sem):
    wid = lax.axis_index("s") * NC + lax.axis_index("c")
    wbase = wid * BPW
    lanes = lax.iota(jnp.int32, L)

    def chunk(g, _):
      base = wbase + g * C
      pltpu.sync_copy(center_hbm.at[pl.ds(base, C)], cidx_v)
      pltpu.sync_copy(context_hbm.at[pl.ds(base, C)], xidx_v)
      for i in range(NIDX_R):
        pltpu.sync_copy(
            negidx_hbm.at[pl.ds(base * N_NEG + i * 128, 128)],
            nidx_v.at[i])
      h1 = pltpu.async_copy(ctab_hbm.at[cidx_v], crow_v, sem)
      h2 = pltpu.async_copy(xtab_hbm.at[xidx_v], xrow_v, sem)
      hn = [
          pltpu.async_copy(xtab_hbm.at[nidx_v.at[i]],
                           nrow_v.at[pl.ds(i * 128, 128)], sem)
          for i in range(NIDX_R)
      ]
      h1.wait()
      h2.wait()
      for h in hn:
        h.wait()

      mask0 = lanes == 0
      zero = jnp.zeros((L,), jnp.float32)
      izero = jnp.zeros((L,), jnp.int32)

      def belem(b, _):
        bvec = izero + b
        cr = [crow_v[b, pl.ds(k * L, L)] for k in range(D // L)]
        xr = [xrow_v[b, pl.ds(k * L, L)] for k in range(D // L)]
        acc = cr[0] * xr[0]
        for k in range(1, D // L):
          acc = acc + cr[k] * xr[k]
        plsc.store_scatter(o1_v, [bvec], zero + jnp.sum(acc), mask=mask0)
        for j in range(N_NEG):
          r = b * N_NEG + j
          acc = cr[0] * nrow_v[r, pl.ds(0, L)]
          for k in range(1, D // L):
            acc = acc + cr[k] * nrow_v[r, pl.ds(k * L, L)]
          plsc.store_scatter(o2_v, [bvec + (j * C)],
                             zero + jnp.sum(acc), mask=mask0)
        return 0

      lax.fori_loop(0, C, belem, 0)

      pltpu.sync_copy(o1_v, out1_hbm.at[pl.ds(base, C)])
      for j in range(N_NEG):
        pltpu.sync_copy(o2_v.at[pl.ds(j * C, C)],
                        out2_hbm.at[pl.ds(j * B + base, C)])
      return 0

    lax.fori_loop(0, G, chunk, 0)

  return sc_kernel


def _tc_loss(d1, d2, total):
  def body(d1_ref, d2_ref, out_ref):
    x1 = d1_ref[...]
    x2 = -d2_ref[...]
    ls1 = jnp.minimum(x1, 0.0) - jnp.log(1.0 + jnp.exp(-jnp.abs(x1)))
    ls2 = jnp.minimum(x2, 0.0) - jnp.log(1.0 + jnp.exp(-jnp.abs(x2)))
    out_ref[0, 0] = -(jnp.sum(ls1) + jnp.sum(ls2)) / total

  out = pl.pallas_call(
      body,
      out_shape=jax.ShapeDtypeStruct((1, 1), jnp.float32),
      out_specs=pl.BlockSpec(memory_space=pltpu.SMEM),
  )(d1, d2)
  return out[0, 0]


def kernel(center, context, negative, center_table, context_table):
  B = center.shape[0]
  negidx = negative.reshape(B * N_NEG)
  dots1, dots2 = _sc_dots(B)(center, context, negidx,
                             center_table, context_table)
  return _tc_loss(dots1.reshape(B // 128, 128),
                  dots2.reshape(N_NEG * B // 128, 128),
                  float(B * (N_NEG + 1)))
